# baseline (speedup 1.0000x reference)
import functools

import jax
import jax.numpy as jnp
from jax import lax
from jax.experimental import pallas as pl
from jax.experimental.pallas import tpu as pltpu
from jax.experimental.pallas import tpu_sc as plsc

_PREC = lax.Precision.DEFAULT
_DN = (((1,), (0,)), ((), ()))


def _dot(a, b, xcast=False):
    if xcast:
        a = a.astype(jnp.bfloat16)
    return lax.dot_general(a, b, _DN, precision=_PREC,
                           preferred_element_type=jnp.float32)


def _ceil8(n):
    return (n + 7) // 8 * 8


def _flat_in(img, offs_hw, m_pad=None):
    n, h, w, c = img.shape
    hp, wp = h + 2, w + 2
    xp = jnp.pad(img, ((0, 0), (1, 1), (1, 1), (0, 0)))
    flat = xp.reshape(n, hp * wp, c)
    max_off = max(du * wp + dv for du, dv in offs_hw)
    m = m_pad if m_pad is not None else h * wp
    L = _ceil8(m + max_off + 8)
    return jnp.pad(flat, ((0, 0), (0, L - hp * wp), (0, 0))), wp


def _unflat(y, ho, wp, wo):
    n = y.shape[0]
    c = y.shape[-1]
    return y.reshape(n, ho, wp, c)[:, :, :wo, :]


def _conv_call(xflat, wt, bias, offs, M, relu_in=False, relu_out=False,
               target=None, wp=None, wo=None, nch=1, m_valid=None):
    N, L, Cin = xflat.shape
    T, _, Cout = wt.shape
    with_loss = target is not None
    assert M % nch == 0
    chunk = M // nch

    def body(x_ref, *refs):
        if with_loss:
            t_ref, w_ref, b_ref, o_ref, ss_ref = refs
        else:
            w_ref, b_ref, o_ref = refs
        base = pl.program_id(1) * chunk
        acc = None
        for t, off in enumerate(offs):
            if nch == 1:
                xs = x_ref[0, off:off + chunk, :]
            else:
                a8, r = (off // 8) * 8, off % 8
                start = pl.multiple_of(base + a8, 8)
                xs = x_ref[0, pl.ds(start, chunk + 8), :][r:r + chunk, :]
            if relu_in:
                xs = jnp.maximum(xs, 0.0)
            p = _dot(xs, w_ref[t])
            acc = p if acc is None else acc + p
        acc = acc + b_ref[...]
        if relu_out:
            acc = jnp.maximum(acc, 0.0)
        o_ref[0] = acc
        if with_loss:
            row = lax.broadcasted_iota(jnp.int32, (chunk, 1), 0) + base
            valid = (row % wp < wo)
            if m_valid is not None:
                valid = valid & (row < m_valid)
            mask = valid.astype(jnp.float32)
            dy = (acc - t_ref[0]) * mask
            s = jnp.sum(dy * dy)

            @pl.when((pl.program_id(0) == 0) & (pl.program_id(1) == 0))
            def _():
                ss_ref[...] = jnp.zeros((8, 128), jnp.float32)

            ss_ref[...] += jnp.full((8, 128), s, jnp.float32)

    in_specs = [pl.BlockSpec((1, L, Cin), lambda i, c: (i, 0, 0))]
    if with_loss:
        in_specs.append(pl.BlockSpec((1, chunk, Cout), lambda i, c: (i, c, 0)))
    in_specs += [
        pl.BlockSpec((T, Cin, Cout), lambda i, c: (0, 0, 0)),
        pl.BlockSpec((1, Cout), lambda i, c: (0, 0)),
    ]
    out_specs = pl.BlockSpec((1, chunk, Cout), lambda i, c: (i, c, 0))
    out_shape = jax.ShapeDtypeStruct((N, M, Cout), jnp.float32)
    if with_loss:
        out_specs = [out_specs, pl.BlockSpec((8, 128), lambda i, c: (0, 0))]
        out_shape = [out_shape, jax.ShapeDtypeStruct((8, 128), jnp.float32)]
    args = (xflat, target, wt, bias) if with_loss else (xflat, wt, bias)
    return pl.pallas_call(
        body,
        grid=(N, nch),
        in_specs=in_specs,
        out_specs=out_specs,
        out_shape=out_shape,
    )(*args)


def _conv_s2_call(xflat, wt, bias, taps, M, cin, relu_out=False, nch=1, xcast=False):
    N, L, Cq = xflat.shape
    T, _, Cout = wt.shape
    assert M % nch == 0
    chunk = M // nch

    def body(x_ref, w_ref, b_ref, o_ref):
        base = pl.program_id(1) * chunk
        acc = None
        for t, (off, lo) in enumerate(taps):
            if nch == 1:
                xs = x_ref[0, off:off + chunk, lo:lo + cin]
            else:
                a8, r = (off // 8) * 8, off % 8
                start = pl.multiple_of(base + a8, 8)
                xs = x_ref[0, pl.ds(start, chunk + 8), lo:lo + cin][r:r + chunk, :]
            p = _dot(xs, w_ref[t], xcast=xcast)
            acc = p if acc is None else acc + p
        acc = acc + b_ref[...]
        if relu_out:
            acc = jnp.maximum(acc, 0.0)
        o_ref[0] = acc

    return pl.pallas_call(
        body,
        grid=(N, nch),
        in_specs=[
            pl.BlockSpec((1, L, Cq), lambda i, c: (i, 0, 0)),
            pl.BlockSpec((T, cin, Cout), lambda i, c: (0, 0, 0)),
            pl.BlockSpec((1, Cout), lambda i, c: (0, 0)),
        ],
        out_specs=pl.BlockSpec((1, chunk, Cout), lambda i, c: (i, c, 0)),
        out_shape=jax.ShapeDtypeStruct((N, M, Cout), jnp.float32),
    )(xflat, wt, bias)


def _w_s2_xla(w):
    t = jnp.transpose(w, (2, 3, 1, 0))
    return t.reshape(16, w.shape[1], w.shape[0])


def _s2_taps(wq, cin):
    return [((kh // 2) * wq + kw // 2, ((kh % 2) * 2 + kw % 2) * cin)
            for kh in range(4) for kw in range(4)]


def _res_call(xflat, wa, wb, offs, M, center):
    N, L, C = xflat.shape
    _, _, Cm = wa.shape

    def body(x_ref, wa_ref, wb_ref, o_ref):
        acc = None
        for t, off in enumerate(offs):
            xs = jnp.maximum(x_ref[0, off:off + M, :], 0.0)
            p = _dot(xs, wa_ref[t])
            acc = p if acc is None else acc + p
        a_bf = acc.astype(jnp.bfloat16)
        h_in = jnp.maximum(a_bf, jnp.bfloat16(0))
        h = _dot(h_in, wb_ref[0])
        o_ref[0] = x_ref[0, center:center + M, :] + h

    return pl.pallas_call(
        body,
        grid=(N,),
        in_specs=[
            pl.BlockSpec((1, L, C), lambda i: (i, 0, 0)),
            pl.BlockSpec((9, C, Cm), lambda i: (0, 0, 0)),
            pl.BlockSpec((1, Cm, C), lambda i: (0, 0, 0)),
        ],
        out_specs=pl.BlockSpec((1, M, C), lambda i: (i, 0, 0)),
        out_shape=jax.ShapeDtypeStruct((N, M, C), jnp.float32),
    )(xflat, wa, wb)


def _vq_call(x, wv, bv, embT):
    N, M, C = x.shape
    D, K = embT.shape

    def body(x_ref, wv_ref, bv_ref, e_ref, zt_ref, idx_ref):
        xr = jnp.maximum(x_ref[0], 0.0)
        flat = _dot(xr, wv_ref[...]) + bv_ref[...]
        embT_v = e_ref[...]
        fsq = jnp.sum(flat * flat, axis=1, keepdims=True)
        esq = jnp.sum(embT_v * embT_v, axis=0, keepdims=True)
        d2 = fsq - 2.0 * _dot(flat, embT_v, xcast=True) + esq
        d = jnp.sqrt(jnp.maximum(d2, 0.0))
        m = jnp.min(d, axis=1, keepdims=True)
        iota = lax.broadcasted_iota(jnp.int32, (M, K), 1)
        idx = jnp.min(jnp.where(d == m, iota, K), axis=1)
        zt_ref[0] = flat
        idx_ref[0, 0] = idx

    return pl.pallas_call(
        body,
        grid=(N,),
        in_specs=[
            pl.BlockSpec((1, M, C), lambda i: (i, 0, 0)),
            pl.BlockSpec((C, D), lambda i: (0, 0)),
            pl.BlockSpec((1, D), lambda i: (0, 0)),
            pl.BlockSpec((D, K), lambda i: (0, 0)),
        ],
        out_specs=[
            pl.BlockSpec((1, M, D), lambda i: (i, 0, 0)),
            pl.BlockSpec((1, 1, M), lambda i: (i, 0, 0)),
        ],
        out_shape=[
            jax.ShapeDtypeStruct((N, M, D), jnp.float32),
            jax.ShapeDtypeStruct((N, 1, M), jnp.int32),
        ],
    )(x, wv, bv, embT)


def _sc_gather(emb, idx):
    K, D = emb.shape
    (B,) = idx.shape
    info = plsc.get_sparse_core_info()
    NC, NS = info.num_cores, info.num_subcores
    NW = NC * NS
    b_per_w = B // NW
    mesh = plsc.VectorSubcoreMesh(core_axis_name="c", subcore_axis_name="s")

    @functools.partial(
        pl.kernel,
        mesh=mesh,
        out_type=jax.ShapeDtypeStruct((B, D), jnp.float32),
        scratch_types=[
            pltpu.VMEM((b_per_w,), jnp.int32),
            pltpu.VMEM((b_per_w, D), jnp.float32),
            pltpu.SemaphoreType.DMA,
        ],
    )
    def k(table_hbm, idx_hbm, out_hbm, idx_v, rows_v, sem):
        wid = lax.axis_index("s") * NC + lax.axis_index("c")
        base = wid * b_per_w
        pltpu.sync_copy(idx_hbm.at[pl.ds(base, b_per_w)], idx_v)
        pltpu.async_copy(table_hbm.at[idx_v], rows_v, sem).wait()
        pltpu.sync_copy(rows_v, out_hbm.at[pl.ds(base, b_per_w)])

    return k(emb, idx)


def _conv4_call(qflat, ztp, wt, bias, offs, M, center):
    N, L, Cin = qflat.shape
    T, _, Cout = wt.shape
    D = ztp.shape[-1]

    def body(q_ref, zt_ref, w_ref, b_ref, o_ref, ss_ref):
        acc = None
        for t, off in enumerate(offs):
            xs = q_ref[0, off:off + M, :]
            p = _dot(xs, w_ref[t])
            acc = p if acc is None else acc + p
        o_ref[0] = acc + b_ref[...]
        dq = q_ref[0, center:center + M, :D] - zt_ref[0]
        s = jnp.sum(dq * dq)

        @pl.when(pl.program_id(0) == 0)
        def _():
            ss_ref[...] = jnp.zeros((8, 128), jnp.float32)

        ss_ref[...] += jnp.full((8, 128), s, jnp.float32)

    return pl.pallas_call(
        body,
        grid=(N,),
        in_specs=[
            pl.BlockSpec((1, L, Cin), lambda i: (i, 0, 0)),
            pl.BlockSpec((1, M, D), lambda i: (i, 0, 0)),
            pl.BlockSpec((T, Cin, Cout), lambda i: (0, 0, 0)),
            pl.BlockSpec((1, Cout), lambda i: (0, 0)),
        ],
        out_specs=[
            pl.BlockSpec((1, M, Cout), lambda i: (i, 0, 0)),
            pl.BlockSpec((8, 128), lambda i: (0, 0)),
        ],
        out_shape=[
            jax.ShapeDtypeStruct((N, M, Cout), jnp.float32),
            jax.ShapeDtypeStruct((8, 128), jnp.float32),
        ],
    )(qflat, ztp, wt, bias)


def _w_s2d(w):
    co, ci = w.shape[0], w.shape[1]
    t = jnp.transpose(w, (2, 3, 1, 0))
    t = t.reshape(2, 2, 2, 2, ci, co)
    t = jnp.transpose(t, (0, 2, 1, 3, 4, 5))
    return t.reshape(4, 4 * ci, co)


def _w_3x3(w):
    co, ci = w.shape[0], w.shape[1]
    return jnp.transpose(w, (2, 3, 1, 0)).reshape(9, ci, co)


def _w_T(w):
    ci, co = w.shape[0], w.shape[1]
    z = jnp.zeros((ci, co), jnp.float32)
    taps = []
    for a in range(3):
        for b in range(3):
            blocks = []
            for r in range(2):
                for s in range(2):
                    kh, kw = 3 - 2 * a + r, 3 - 2 * b + s
                    ok = 0 <= kh < 4 and 0 <= kw < 4
                    blocks.append(w[:, :, kh, kw] if ok else z)
            taps.append(jnp.concatenate(blocks, axis=1))
    return jnp.stack(taps)


def _s2d(x):
    n, h, w, c = x.shape
    x = x.reshape(n, h // 2, 2, w // 2, 2, c)
    x = jnp.transpose(x, (0, 1, 3, 2, 4, 5))
    return x.reshape(n, h // 2, w // 2, 4 * c)


def _d2s(x, cout):
    n, h, w, _ = x.shape
    x = x.reshape(n, h, w, 2, 2, cout)
    x = jnp.transpose(x, (0, 1, 3, 2, 4, 5))
    return x.reshape(n, 2 * h, 2 * w, cout)


def _flat_s2d(xq, offs, m_pad=None):
    n, hq, wq, c = xq.shape
    flat = xq.reshape(n, hq * wq, c)
    max_off = max(offs)
    m = m_pad if m_pad is not None else (hq - 1) * wq
    L = _ceil8(m + max_off)
    return jnp.pad(flat, ((0, 0), (0, L - hq * wq), (0, 0)))


def kernel(inputs, params):
    p = params
    N = inputs.shape[0]
    offs2 = [(0, 0), (0, 1), (1, 0), (1, 1)]
    offs3 = [(du, dv) for du in range(3) for dv in range(3)]

    xin = jnp.transpose(inputs, (0, 2, 3, 1))
    xq = _s2d(jnp.pad(xin, ((0, 0), (1, 1), (1, 1), (0, 0))))
    o1 = [du * 113 + dv for du, dv in offs2]
    x1 = _conv_s2_call(_flat_s2d(xq, o1, 12672),
                       _w_s2_xla(p['conv1_w']).astype(jnp.bfloat16),
                       p['conv1_b'][None], _s2_taps(113, 3), 12672, 3,
                       relu_out=True, nch=16)
    x1i = _unflat(x1[:, :112 * 113], 112, 113, 112)
    xq2 = _s2d(jnp.pad(x1i, ((0, 0), (1, 1), (1, 1), (0, 0))))
    o2 = [du * 57 + dv for du, dv in offs2]
    x2 = _conv_s2_call(_flat_s2d(xq2, o2),
                       _w_s2_xla(p['conv2_w']).astype(jnp.bfloat16),
                       p['conv2_b'][None], _s2_taps(57, 64), 56 * 57, 64,
                       relu_out=True, xcast=True)
    x2i = _unflat(x2, 56, 57, 56)

    o3 = [du * 58 + dv for du, dv in offs3]
    M = 56 * 58
    ctr = 59
    xf, _ = _flat_in(x2i, offs3)
    x = _conv_call(xf, _w_3x3(p['conv3_w']), p['conv3_b'][None], o3, M)
    for i in range(p['res1_a_w'].shape[0]):
        xf, _ = _flat_in(_unflat(x, 56, 58, 56), offs3)
        x = _res_call(xf, _w_3x3(p['res1_a_w'][i]).astype(jnp.bfloat16),
                      jnp.transpose(p['res1_b_w'][i, :, :, 0, 0],
                                    (1, 0))[None], o3, M, ctr)

    emb = p['emb']
    K, D = emb.shape
    wv = jnp.transpose(p['vq_w'][:, :, 0, 0], (1, 0))
    xv = _unflat(x, 56, 58, 56).reshape(N, 56 * 56, 128)
    zt, idx = _vq_call(xv, wv, p['vq_b'][None], jnp.transpose(emb, (1, 0)))
    emb_p = jnp.pad(emb, ((0, 0), (0, 128 - D)))
    quant = _sc_gather(emb_p, idx.reshape(-1))
    quant = quant.reshape(N, 56, 56, 128)

    w4 = jnp.pad(_w_3x3(p['conv4_w']), ((0, 0), (0, 128 - D), (0, 0)))
    qf, _ = _flat_in(quant, offs3)
    ztp = jnp.pad(zt.reshape(N, 56, 56, D),
                  ((0, 0), (0, 0), (0, 2), (0, 0))).reshape(N, M, D)
    y4, vq_ss = _conv4_call(qf, ztp, w4, p['conv4_b'][None], o3, M, ctr)
    y = y4
    for i in range(p['res2_a_w'].shape[0]):
        yf, _ = _flat_in(_unflat(y, 56, 58, 56), offs3)
        y = _res_call(yf, _w_3x3(p['res2_a_w'][i]).astype(jnp.bfloat16),
                      jnp.transpose(p['res2_b_w'][i, :, :, 0, 0],
                                    (1, 0))[None], o3, M, ctr)

    yf, _ = _flat_in(_unflat(y, 56, 58, 56), offs3)
    y5p = _conv_call(yf, _w_T(p['conv5_w']).astype(jnp.bfloat16),
                     jnp.tile(p['conv5_b'], 4)[None],
                     o3, M, relu_in=True)
    y5 = _d2s(_unflat(y5p, 56, 58, 56), 32)
    inp_par = _s2d(xin)
    inp_flat = jnp.pad(inp_par, ((0, 0), (0, 0), (0, 2), (0, 0)))
    inp_flat = jnp.pad(inp_flat.reshape(N, 112 * 114, 12),
                       ((0, 0), (0, 12800 - 112 * 114), (0, 0)))
    o6 = [du * 114 + dv for du, dv in offs3]
    y5f, _ = _flat_in(y5, offs3, m_pad=12800)
    y6p, rec_ss = _conv_call(y5f, _w_T(p['conv6_w']).astype(jnp.bfloat16),
                             jnp.tile(p['conv6_b'], 4)[None], o6, 12800,
                             relu_in=True, target=inp_flat, wp=114, wo=112,
                             nch=16, m_valid=112 * 114)
    y_out = jnp.transpose(_d2s(_unflat(y6p[:, :112 * 114], 112, 114, 112), 3),
                          (0, 3, 1, 2))

    m = jnp.float32(N * 56 * 56 * D)
    c_loss = 1.25 * vq_ss[0, 0] / m
    rec_loss = rec_ss[0, 0] / jnp.float32(inputs.size)
    loss = rec_loss + c_loss
    return (loss, y_out, rec_loss)
